# baseline (speedup 1.0000x reference)
import functools
import jax
import jax.numpy as jnp
from jax import lax
from jax.experimental import pallas as pl
from jax.experimental.pallas import tpu as pltpu, tpu_sc as plsc

NC = 2
NS = 16
NW = NC * NS
CH = 128


def _sc_deg(n_pad, ce):
    rows_per_tile = n_pad // NS
    mesh = plsc.VectorSubcoreMesh(core_axis_name="c", subcore_axis_name="s")

    @functools.partial(
        pl.kernel,
        out_type=jax.ShapeDtypeStruct((NC * n_pad,), jnp.float32),
        mesh=mesh,
        scratch_types=[
            pltpu.VMEM((ce,), jnp.int32),
            pltpu.VMEM((n_pad,), jnp.float32),
            pltpu.VMEM((rows_per_tile,), jnp.float32),
            pltpu.VMEM((rows_per_tile,), jnp.float32),
            pltpu.VMEM_SHARED((NS, n_pad), jnp.float32),
        ],
        compiler_params=pltpu.CompilerParams(needs_layout_passes=False),
    )
    def deg_kernel(dst_hbm, out_hbm, idx_v, hist, accum, tmp, shared):
        c = lax.axis_index("c")
        s = lax.axis_index("s")
        wid = s * NC + c
        zeros16 = jnp.zeros((16,), jnp.float32)
        ones16 = jnp.ones((16,), jnp.float32)

        @pl.loop(0, n_pad // 16)
        def _(i):
            hist[pl.ds(i * 16, 16)] = zeros16

        pltpu.sync_copy(dst_hbm.at[wid], idx_v)

        @pl.loop(0, ce // 16)
        def _(i):
            plsc.addupdate_scatter(hist, [idx_v[pl.ds(i * 16, 16)]], ones16)

        pltpu.sync_copy(hist, shared.at[s])
        plsc.subcore_barrier()

        @pl.loop(0, rows_per_tile // 16)
        def _(i):
            accum[pl.ds(i * 16, 16)] = zeros16

        for t in range(NS):
            pltpu.sync_copy(shared.at[t, pl.ds(s * rows_per_tile,
                                               rows_per_tile)], tmp)

            @pl.loop(0, rows_per_tile // 16)
            def _(i):
                sl = pl.ds(i * 16, 16)
                accum[sl] = accum[sl] + tmp[sl]

        pltpu.sync_copy(accum,
                        out_hbm.at[pl.ds(c * n_pad + s * rows_per_tile,
                                         rows_per_tile)])

    return deg_kernel


def _sc_agg(n_pad, nch, d, k0=None, k1=None):
    if k0 is None:
        k0 = k1 = nch
    kmax = max(k0, k1)
    rows_per_tile = n_pad // NS
    mesh = plsc.VectorSubcoreMesh(core_axis_name="c", subcore_axis_name="s")

    @functools.partial(
        pl.kernel,
        out_type=jax.ShapeDtypeStruct((NC * n_pad, d), jnp.float32),
        mesh=mesh,
        scratch_types=[
            pltpu.VMEM((2, CH), jnp.int32),
            pltpu.VMEM((kmax, CH), jnp.int32),
            pltpu.VMEM((2, CH, d), jnp.float32),
            pltpu.VMEM_SHARED((n_pad, d), jnp.float32),
            pltpu.SemaphoreType.DMA((2,)),
            pltpu.SemaphoreType.DMA((2,)),
        ],
    )
    def agg_kernel(hs_hbm, src_hbm, dst_hbm, zeros_hbm, out_hbm,
                   idx_s, idx_d, rows_v, acc, sems, isems):
        c = lax.axis_index("c")
        s = lax.axis_index("s")
        wid = s * NC + c
        pltpu.sync_copy(zeros_hbm.at[pl.ds(s * rows_per_tile, rows_per_tile)],
                        acc.at[pl.ds(s * rows_per_tile, rows_per_tile)])
        pltpu.sync_copy(dst_hbm.at[wid], idx_d)
        plsc.subcore_barrier()

        pltpu.sync_copy(src_hbm.at[wid, 0], idx_s.at[0])
        pltpu.async_copy(hs_hbm.at[idx_s.at[0]], rows_v.at[0], sems.at[0])
        pltpu.sync_copy(src_hbm.at[wid, 1], idx_s.at[1])

        @pl.loop(0, nch - 1)
        def _(j):
            slot = lax.rem(j, 2)
            nslot = lax.rem(j + 1, 2)
            pltpu.make_async_copy(hs_hbm.at[idx_s.at[slot]], rows_v.at[slot],
                                  sems.at[slot]).wait()
            pltpu.async_copy(hs_hbm.at[idx_s.at[nslot]], rows_v.at[nslot],
                             sems.at[nslot])
            pltpu.sync_copy(rows_v.at[slot], acc.at[idx_d.at[j]], add=True)
            pltpu.sync_copy(src_hbm.at[wid, j + 2], idx_s.at[slot])

        last = lax.rem(nch - 1, 2)
        pltpu.make_async_copy(hs_hbm.at[idx_s.at[last]], rows_v.at[last],
                              sems.at[last]).wait()
        pltpu.sync_copy(rows_v.at[last], acc.at[idx_d.at[nch - 1]], add=True)

        plsc.subcore_barrier()
        pltpu.sync_copy(acc.at[pl.ds(s * rows_per_tile, rows_per_tile)],
                        out_hbm.at[pl.ds(c * n_pad + s * rows_per_tile,
                                         rows_per_tile)])

    return agg_kernel


def _tc_matmul(n_pad, f, k, bm):
    def body(x_ref, w_ref, b_ref, out_ref):
        out_ref[...] = jnp.dot(x_ref[...], w_ref[...],
                               preferred_element_type=jnp.float32) + b_ref[...]

    grid = (n_pad // bm,)
    return pl.pallas_call(
        body,
        grid=grid,
        in_specs=[
            pl.BlockSpec((bm, f), lambda i: (i, 0)),
            pl.BlockSpec((f, k), lambda i: (0, 0)),
            pl.BlockSpec((1, k), lambda i: (0, 0)),
        ],
        out_specs=pl.BlockSpec((bm, k), lambda i: (i, 0)),
        out_shape=jax.ShapeDtypeStruct((n_pad, k), jnp.float32),
    )


def _tc_scale(n_pad, h, bm):
    def body(h1_ref, p0_ref, p1_ref, hs_ref, dis_ref):
        deg = p0_ref[:, 0:1] + p1_ref[:, 0:1] + 1.0
        dis = lax.rsqrt(jnp.maximum(deg, 1.0))
        hs_ref[...] = h1_ref[...] * dis
        dis_ref[...] = jnp.broadcast_to(dis, (bm, 8))

    grid = (n_pad // bm,)
    return pl.pallas_call(
        body,
        grid=grid,
        in_specs=[
            pl.BlockSpec((bm, h), lambda i: (i, 0)),
            pl.BlockSpec((bm, 8), lambda i: (i, 0)),
            pl.BlockSpec((bm, 8), lambda i: (i, 0)),
        ],
        out_specs=[
            pl.BlockSpec((bm, h), lambda i: (i, 0)),
            pl.BlockSpec((bm, 8), lambda i: (i, 0)),
        ],
        out_shape=[
            jax.ShapeDtypeStruct((n_pad, h), jnp.float32),
            jax.ShapeDtypeStruct((n_pad, 8), jnp.float32),
        ],
    )


def _tc_layer2(n_pad, h, bm):
    def body(hs_ref, p0_ref, p1_ref, dis_ref, b1_ref, w_ref, out_ref):
        dis = dis_ref[:, 0:1]
        agg = (p0_ref[...] + p1_ref[...] + hs_ref[...]) * dis + b1_ref[...]
        z1 = jnp.maximum(agg, 0.0)
        out_ref[...] = jnp.dot(z1, w_ref[...],
                               preferred_element_type=jnp.float32) * dis

    grid = (n_pad // bm,)
    return pl.pallas_call(
        body,
        grid=grid,
        in_specs=[
            pl.BlockSpec((bm, h), lambda i: (i, 0)),
            pl.BlockSpec((bm, h), lambda i: (i, 0)),
            pl.BlockSpec((bm, h), lambda i: (i, 0)),
            pl.BlockSpec((bm, 8), lambda i: (i, 0)),
            pl.BlockSpec((1, h), lambda i: (0, 0)),
            pl.BlockSpec((h, h), lambda i: (0, 0)),
        ],
        out_specs=pl.BlockSpec((bm, h), lambda i: (i, 0)),
        out_shape=jax.ShapeDtypeStruct((n_pad, h), jnp.float32),
    )


def _tc_gru_head(n_pad, h, bm):
    def body(hs_ref, p0_ref, p1_ref, dis_ref, bvec_ref, h0_ref,
             wih_ref, gh_ref, bih_ref, whd_ref, bhd_ref,
             hn_ref, th_ref):
        dis = dis_ref[:, 0:1]
        z = (p0_ref[...] + p1_ref[...] + hs_ref[...]) * dis + bvec_ref[...]
        h0 = h0_ref[...]
        gi = jnp.dot(z, wih_ref[...],
                     preferred_element_type=jnp.float32) + bih_ref[...]
        gh = gh_ref[...]
        r = jax.nn.sigmoid(gi[:, :h] + gh[:, :h])
        zz = jax.nn.sigmoid(gi[:, h:2 * h] + gh[:, h:2 * h])
        nn_ = jnp.tanh(gi[:, 2 * h:] + r * gh[:, 2 * h:])
        h_next = (1.0 - zz) * nn_ + zz * h0
        hn_ref[...] = h_next
        th_ref[...] = jnp.dot(h_next, whd_ref[...],
                              preferred_element_type=jnp.float32) + bhd_ref[...]

    grid = (n_pad // bm,)
    return pl.pallas_call(
        body,
        grid=grid,
        in_specs=[
            pl.BlockSpec((bm, h), lambda i: (i, 0)),
            pl.BlockSpec((bm, h), lambda i: (i, 0)),
            pl.BlockSpec((bm, h), lambda i: (i, 0)),
            pl.BlockSpec((bm, 8), lambda i: (i, 0)),
            pl.BlockSpec((1, h), lambda i: (0, 0)),
            pl.BlockSpec((bm, h), lambda i: (i, 0)),
            pl.BlockSpec((h, 3 * h), lambda i: (0, 0)),
            pl.BlockSpec((bm, 3 * h), lambda i: (i, 0)),
            pl.BlockSpec((1, 3 * h), lambda i: (0, 0)),
            pl.BlockSpec((h, 8), lambda i: (0, 0)),
            pl.BlockSpec((1, 8), lambda i: (0, 0)),
        ],
        out_specs=[
            pl.BlockSpec((bm, h), lambda i: (i, 0)),
            pl.BlockSpec((bm, 8), lambda i: (i, 0)),
        ],
        out_shape=[
            jax.ShapeDtypeStruct((n_pad, h), jnp.float32),
            jax.ShapeDtypeStruct((n_pad, 8), jnp.float32),
        ],
    )


@jax.jit
def _run(g, x, t_over_t, h_prev, W1, b1, W2, b2, Wt, bt,
         W_ih, W_hh, b_ih, b_hh, W_head, b_head):
    n, f = x.shape
    h = W1.shape[1]
    e = g.shape[1]

    n_pad = ((n + 1 + 1023) // 1024) * 1024
    ce = ((e + NW - 1) // NW + CH - 1) // CH * CH
    nch = ce // CH
    e_pad = ce * NW

    src = g[0].astype(jnp.int32)
    dst = g[1].astype(jnp.int32)
    fill = jnp.full((e_pad - e,), n, dtype=jnp.int32)
    src_p = jnp.concatenate([src, fill])
    dst_p = jnp.concatenate([dst, fill])
    dst_even = dst_p.reshape(NW, ce)

    k0 = k1 = nch
    kmax = max(k0, k1)

    def skew(ep, extra):
        p0 = ep[:NS * k0 * CH].reshape(NS, k0, CH)
        p1 = ep[NS * k0 * CH:].reshape(NS, k1, CH)
        w = jnp.full((NS, NC, kmax + extra, CH), n, dtype=jnp.int32)
        w = w.at[:, 0, :k0].set(p0).at[:, 1, :k1].set(p1)
        return w.reshape(NW, kmax + extra, CH)

    src_r = skew(src_p, 1)
    dst_r = skew(dst_p, 0)

    x_pad = jnp.zeros((n_pad, f), x.dtype).at[:n].set(x)
    h0_pad = jnp.zeros((n_pad, h), x.dtype).at[:n].set(h_prev[0])
    zeros_big = jnp.zeros((n_pad, h), jnp.float32)

    deg_parts = _sc_deg(n_pad, ce)(dst_even)
    dp0 = jnp.broadcast_to(deg_parts[:n_pad, None], (n_pad, 8))
    dp1 = jnp.broadcast_to(deg_parts[n_pad:, None], (n_pad, 8))

    zero_h = jnp.zeros((1, h), jnp.float32)
    h1 = _tc_matmul(n_pad, f, h, 1024)(x_pad, W1, zero_h)
    hs1, dis = _tc_scale(n_pad, h, 1024)(h1, dp0, dp1)

    agg = _sc_agg(n_pad, nch, h, k0, k1)
    parts1 = agg(hs1, src_r, dst_r, zeros_big)
    hs2 = _tc_layer2(n_pad, h, 1024)(
        hs1, parts1[:n_pad], parts1[n_pad:], dis, b1.reshape(1, h), W2)

    parts2 = agg(hs2, src_r, dst_r, zeros_big)

    gh = _tc_matmul(n_pad, h, 3 * h, 1024)(h0_pad, W_hh.T,
                                           b_hh.reshape(1, 3 * h))

    bvec = (b2 + bt + t_over_t[0] * Wt[:, 0]).reshape(1, h)
    whd = jnp.zeros((h, 8), jnp.float32).at[:, :3].set(W_head.T)
    bhd = jnp.zeros((1, 8), jnp.float32).at[0, :3].set(b_head)
    h_next_pad, theta_pad = _tc_gru_head(n_pad, h, 1024)(
        hs2, parts2[:n_pad], parts2[n_pad:], dis, bvec, h0_pad,
        W_ih.T, gh, b_ih.reshape(1, 3 * h),
        whd, bhd)

    return theta_pad[:n, :3], h_next_pad[:n]


def kernel(g, x, t_over_t, h_prev, W1, b1, W2, b2, Wt, bt,
           W_ih, W_hh, b_ih, b_hh, W_head, b_head):
    return _run(g, x, t_over_t, h_prev, W1, b1, W2, b2, Wt, bt,
                W_ih, W_hh, b_ih, b_hh, W_head, b_head)


# speedup vs baseline: 1.1590x; 1.1590x over previous
import functools
import jax
import jax.numpy as jnp
from jax import lax
from jax.experimental import pallas as pl
from jax.experimental.pallas import tpu as pltpu, tpu_sc as plsc

NC = 2
NS = 16
NW = NC * NS
CH = 128


def _sc_deg(n_pad, ce):
    rows_per_tile = n_pad // NS
    mesh = plsc.VectorSubcoreMesh(core_axis_name="c", subcore_axis_name="s")

    @functools.partial(
        pl.kernel,
        out_type=jax.ShapeDtypeStruct((NC * n_pad,), jnp.float32),
        mesh=mesh,
        scratch_types=[
            pltpu.VMEM((ce,), jnp.int32),
            pltpu.VMEM((n_pad,), jnp.float32),
            pltpu.VMEM((rows_per_tile,), jnp.float32),
            pltpu.VMEM((rows_per_tile,), jnp.float32),
            pltpu.VMEM_SHARED((NS, n_pad), jnp.float32),
        ],
        compiler_params=pltpu.CompilerParams(needs_layout_passes=False),
    )
    def deg_kernel(dst_hbm, out_hbm, idx_v, hist, accum, tmp, shared):
        c = lax.axis_index("c")
        s = lax.axis_index("s")
        wid = s * NC + c
        zeros16 = jnp.zeros((16,), jnp.float32)
        ones16 = jnp.ones((16,), jnp.float32)

        @pl.loop(0, n_pad // 16)
        def _(i):
            hist[pl.ds(i * 16, 16)] = zeros16

        pltpu.sync_copy(dst_hbm.at[wid], idx_v)

        @pl.loop(0, ce // 16)
        def _(i):
            plsc.addupdate_scatter(hist, [idx_v[pl.ds(i * 16, 16)]], ones16)

        pltpu.sync_copy(hist, shared.at[s])
        plsc.subcore_barrier()

        @pl.loop(0, rows_per_tile // 16)
        def _(i):
            accum[pl.ds(i * 16, 16)] = zeros16

        for t in range(NS):
            pltpu.sync_copy(shared.at[t, pl.ds(s * rows_per_tile,
                                               rows_per_tile)], tmp)

            @pl.loop(0, rows_per_tile // 16)
            def _(i):
                sl = pl.ds(i * 16, 16)
                accum[sl] = accum[sl] + tmp[sl]

        pltpu.sync_copy(accum,
                        out_hbm.at[pl.ds(c * n_pad + s * rows_per_tile,
                                         rows_per_tile)])

    return deg_kernel


def _sc_agg(n_pad, nch, d, k0=None, k1=None):
    if k0 is None:
        k0 = k1 = nch
    kmax = max(k0, k1)
    rows_per_tile = n_pad // NS
    mesh = plsc.VectorSubcoreMesh(core_axis_name="c", subcore_axis_name="s")

    @functools.partial(
        pl.kernel,
        out_type=jax.ShapeDtypeStruct((NC * n_pad, d), jnp.float32),
        mesh=mesh,
        scratch_types=[
            pltpu.VMEM((2, CH), jnp.int32),
            pltpu.VMEM((kmax, CH), jnp.int32),
            pltpu.VMEM((2, CH, d), jnp.float32),
            pltpu.VMEM_SHARED((n_pad, d), jnp.float32),
            pltpu.SemaphoreType.DMA((2,)),
            pltpu.SemaphoreType.DMA((2,)),
        ],
    )
    def agg_kernel(hs_hbm, src_hbm, dst_hbm, zeros_hbm, out_hbm,
                   idx_s, idx_d, rows_v, acc, sems, isems):
        c = lax.axis_index("c")
        s = lax.axis_index("s")
        wid = s * NC + c
        pltpu.sync_copy(zeros_hbm.at[pl.ds(s * rows_per_tile, rows_per_tile)],
                        acc.at[pl.ds(s * rows_per_tile, rows_per_tile)])
        pltpu.sync_copy(dst_hbm.at[wid], idx_d)
        plsc.subcore_barrier()

        pltpu.sync_copy(src_hbm.at[wid, 0], idx_s.at[0])
        pltpu.async_copy(hs_hbm.at[idx_s.at[0]], rows_v.at[0], sems.at[0])
        pltpu.sync_copy(src_hbm.at[wid, 1], idx_s.at[1])

        @pl.loop(0, nch - 1)
        def _(j):
            slot = lax.rem(j, 2)
            nslot = lax.rem(j + 1, 2)
            pltpu.make_async_copy(hs_hbm.at[idx_s.at[slot]], rows_v.at[slot],
                                  sems.at[slot]).wait()
            pltpu.async_copy(hs_hbm.at[idx_s.at[nslot]], rows_v.at[nslot],
                             sems.at[nslot])
            pltpu.sync_copy(rows_v.at[slot], acc.at[idx_d.at[j]], add=True)
            pltpu.sync_copy(src_hbm.at[wid, j + 2], idx_s.at[slot])

        last = lax.rem(nch - 1, 2)
        pltpu.make_async_copy(hs_hbm.at[idx_s.at[last]], rows_v.at[last],
                              sems.at[last]).wait()
        pltpu.sync_copy(rows_v.at[last], acc.at[idx_d.at[nch - 1]], add=True)

        plsc.subcore_barrier()
        pltpu.sync_copy(acc.at[pl.ds(s * rows_per_tile, rows_per_tile)],
                        out_hbm.at[pl.ds(c * n_pad + s * rows_per_tile,
                                         rows_per_tile)])

    return agg_kernel


def _tc_matmul(n_pad, f, k, bm):
    def body(x_ref, w_ref, b_ref, out_ref):
        out_ref[...] = jnp.dot(x_ref[...], w_ref[...],
                               preferred_element_type=jnp.float32) + b_ref[...]

    grid = (n_pad // bm,)
    return pl.pallas_call(
        body,
        grid=grid,
        in_specs=[
            pl.BlockSpec((bm, f), lambda i: (i, 0)),
            pl.BlockSpec((f, k), lambda i: (0, 0)),
            pl.BlockSpec((1, k), lambda i: (0, 0)),
        ],
        out_specs=pl.BlockSpec((bm, k), lambda i: (i, 0)),
        out_shape=jax.ShapeDtypeStruct((n_pad, k), jnp.float32),
    )


def _tc_scale(n_pad, h, bm):
    def body(h1_ref, p0_ref, p1_ref, hs_ref, dis_ref):
        deg = p0_ref[:, 0:1] + p1_ref[:, 0:1] + 1.0
        dis = lax.rsqrt(jnp.maximum(deg, 1.0))
        hs_ref[...] = h1_ref[...] * dis
        dis_ref[...] = jnp.broadcast_to(dis, (bm, 8))

    grid = (n_pad // bm,)
    return pl.pallas_call(
        body,
        grid=grid,
        in_specs=[
            pl.BlockSpec((bm, h), lambda i: (i, 0)),
            pl.BlockSpec((bm, 8), lambda i: (i, 0)),
            pl.BlockSpec((bm, 8), lambda i: (i, 0)),
        ],
        out_specs=[
            pl.BlockSpec((bm, h), lambda i: (i, 0)),
            pl.BlockSpec((bm, 8), lambda i: (i, 0)),
        ],
        out_shape=[
            jax.ShapeDtypeStruct((n_pad, h), jnp.float32),
            jax.ShapeDtypeStruct((n_pad, 8), jnp.float32),
        ],
    )


def _tc_layer2(n_pad, h, bm):
    def body(hs_ref, p0_ref, p1_ref, dis_ref, b1_ref, w_ref, out_ref):
        dis = dis_ref[:, 0:1]
        agg = (p0_ref[...] + p1_ref[...] + hs_ref[...]) * dis + b1_ref[...]
        z1 = jnp.maximum(agg, 0.0)
        out_ref[...] = jnp.dot(z1, w_ref[...],
                               preferred_element_type=jnp.float32) * dis

    grid = (n_pad // bm,)
    return pl.pallas_call(
        body,
        grid=grid,
        in_specs=[
            pl.BlockSpec((bm, h), lambda i: (i, 0)),
            pl.BlockSpec((bm, h), lambda i: (i, 0)),
            pl.BlockSpec((bm, h), lambda i: (i, 0)),
            pl.BlockSpec((bm, 8), lambda i: (i, 0)),
            pl.BlockSpec((1, h), lambda i: (0, 0)),
            pl.BlockSpec((h, h), lambda i: (0, 0)),
        ],
        out_specs=pl.BlockSpec((bm, h), lambda i: (i, 0)),
        out_shape=jax.ShapeDtypeStruct((n_pad, h), jnp.float32),
    )


def _tc_gru_head(n_pad, h, bm):
    def body(hs_ref, p0_ref, p1_ref, dis_ref, bvec_ref, h0_ref,
             wih_ref, gh_ref, bih_ref, whd_ref, bhd_ref,
             hn_ref, th_ref):
        dis = dis_ref[:, 0:1]
        z = (p0_ref[...] + p1_ref[...] + hs_ref[...]) * dis + bvec_ref[...]
        h0 = h0_ref[...]
        gi = jnp.dot(z, wih_ref[...],
                     preferred_element_type=jnp.float32) + bih_ref[...]
        gh = gh_ref[...]
        r = jax.nn.sigmoid(gi[:, :h] + gh[:, :h])
        zz = jax.nn.sigmoid(gi[:, h:2 * h] + gh[:, h:2 * h])
        nn_ = jnp.tanh(gi[:, 2 * h:] + r * gh[:, 2 * h:])
        h_next = (1.0 - zz) * nn_ + zz * h0
        hn_ref[...] = h_next
        th_ref[...] = jnp.dot(h_next, whd_ref[...],
                              preferred_element_type=jnp.float32) + bhd_ref[...]

    grid = (n_pad // bm,)
    return pl.pallas_call(
        body,
        grid=grid,
        in_specs=[
            pl.BlockSpec((bm, h), lambda i: (i, 0)),
            pl.BlockSpec((bm, h), lambda i: (i, 0)),
            pl.BlockSpec((bm, h), lambda i: (i, 0)),
            pl.BlockSpec((bm, 8), lambda i: (i, 0)),
            pl.BlockSpec((1, h), lambda i: (0, 0)),
            pl.BlockSpec((bm, h), lambda i: (i, 0)),
            pl.BlockSpec((h, 3 * h), lambda i: (0, 0)),
            pl.BlockSpec((bm, 3 * h), lambda i: (i, 0)),
            pl.BlockSpec((1, 3 * h), lambda i: (0, 0)),
            pl.BlockSpec((h, 8), lambda i: (0, 0)),
            pl.BlockSpec((1, 8), lambda i: (0, 0)),
        ],
        out_specs=[
            pl.BlockSpec((bm, h), lambda i: (i, 0)),
            pl.BlockSpec((bm, 8), lambda i: (i, 0)),
        ],
        out_shape=[
            jax.ShapeDtypeStruct((n_pad, h), jnp.float32),
            jax.ShapeDtypeStruct((n_pad, 8), jnp.float32),
        ],
    )


@jax.jit
def _run(g, x, t_over_t, h_prev, W1, b1, W2, b2, Wt, bt,
         W_ih, W_hh, b_ih, b_hh, W_head, b_head):
    n, f = x.shape
    h = W1.shape[1]
    e = g.shape[1]

    n_pad = ((n + 1 + 1023) // 1024) * 1024
    ce = ((e + NW - 1) // NW + CH - 1) // CH * CH
    nch = ce // CH
    e_pad = ce * NW

    src = g[0].astype(jnp.int32)
    dst = g[1].astype(jnp.int32)
    fill = jnp.full((e_pad - e,), n, dtype=jnp.int32)
    src_p = jnp.concatenate([src, fill])
    dst_p = jnp.concatenate([dst, fill])
    dst_even = dst_p.reshape(NW, ce)
    k0 = k1 = nch

    src_r = src_p.reshape(NW, nch, CH)
    src_r = jnp.concatenate(
        [src_r, jnp.full((NW, 1, CH), n, dtype=jnp.int32)], axis=1)
    dst_r = dst_p.reshape(NW, nch, CH)

    x_pad = jnp.zeros((n_pad, f), x.dtype).at[:n].set(x)
    h0_pad = jnp.zeros((n_pad, h), x.dtype).at[:n].set(h_prev[0])
    zeros_big = jnp.zeros((n_pad, h), jnp.float32)

    deg_parts = _sc_deg(n_pad, ce)(dst_even)
    dp0 = jnp.broadcast_to(deg_parts[:n_pad, None], (n_pad, 8))
    dp1 = jnp.broadcast_to(deg_parts[n_pad:, None], (n_pad, 8))

    zero_h = jnp.zeros((1, h), jnp.float32)
    h1 = _tc_matmul(n_pad, f, h, 1024)(x_pad, W1, zero_h)
    hs1, dis = _tc_scale(n_pad, h, 1024)(h1, dp0, dp1)

    agg = _sc_agg(n_pad, nch, h, k0, k1)
    parts1 = agg(hs1, src_r, dst_r, zeros_big)
    hs2 = _tc_layer2(n_pad, h, 1024)(
        hs1, parts1[:n_pad], parts1[n_pad:], dis, b1.reshape(1, h), W2)

    parts2 = agg(hs2, src_r, dst_r, zeros_big)

    gh = _tc_matmul(n_pad, h, 3 * h, 1024)(h0_pad, W_hh.T,
                                           b_hh.reshape(1, 3 * h))

    bvec = (b2 + bt + t_over_t[0] * Wt[:, 0]).reshape(1, h)
    whd = jnp.zeros((h, 8), jnp.float32).at[:, :3].set(W_head.T)
    bhd = jnp.zeros((1, 8), jnp.float32).at[0, :3].set(b_head)
    h_next_pad, theta_pad = _tc_gru_head(n_pad, h, 1024)(
        hs2, parts2[:n_pad], parts2[n_pad:], dis, bvec, h0_pad,
        W_ih.T, gh, b_ih.reshape(1, 3 * h),
        whd, bhd)

    return theta_pad[:n, :3], h_next_pad[:n]


def kernel(g, x, t_over_t, h_prev, W1, b1, W2, b2, Wt, bt,
           W_ih, W_hh, b_ih, b_hh, W_head, b_head):
    return _run(g, x, t_over_t, h_prev, W1, b1, W2, b2, Wt, bt,
                W_ih, W_hh, b_ih, b_hh, W_head, b_head)


# speedup vs baseline: 1.1594x; 1.0003x over previous
import functools
import jax
import jax.numpy as jnp
from jax import lax
from jax.experimental import pallas as pl
from jax.experimental.pallas import tpu as pltpu, tpu_sc as plsc

NC = 2
NS = 16
NW = NC * NS
CH = 128


def _sc_deg(n_pad, ce):
    rows_per_tile = n_pad // NS
    mesh = plsc.VectorSubcoreMesh(core_axis_name="c", subcore_axis_name="s")

    @functools.partial(
        pl.kernel,
        out_type=jax.ShapeDtypeStruct((NC * n_pad,), jnp.float32),
        mesh=mesh,
        scratch_types=[
            pltpu.VMEM((ce,), jnp.int32),
            pltpu.VMEM((n_pad,), jnp.float32),
            pltpu.VMEM((rows_per_tile,), jnp.float32),
            pltpu.VMEM((rows_per_tile,), jnp.float32),
            pltpu.VMEM_SHARED((NS, n_pad), jnp.float32),
        ],
        compiler_params=pltpu.CompilerParams(needs_layout_passes=False),
    )
    def deg_kernel(dst_hbm, out_hbm, idx_v, hist, accum, tmp, shared):
        c = lax.axis_index("c")
        s = lax.axis_index("s")
        wid = s * NC + c
        zeros16 = jnp.zeros((16,), jnp.float32)
        ones16 = jnp.ones((16,), jnp.float32)

        @pl.loop(0, n_pad // 16)
        def _(i):
            hist[pl.ds(i * 16, 16)] = zeros16

        pltpu.sync_copy(dst_hbm.at[wid], idx_v)

        @pl.loop(0, ce // 16)
        def _(i):
            plsc.addupdate_scatter(hist, [idx_v[pl.ds(i * 16, 16)]], ones16)

        pltpu.sync_copy(hist, shared.at[s])
        plsc.subcore_barrier()

        @pl.loop(0, rows_per_tile // 16)
        def _(i):
            accum[pl.ds(i * 16, 16)] = zeros16

        for t in range(NS):
            pltpu.sync_copy(shared.at[t, pl.ds(s * rows_per_tile,
                                               rows_per_tile)], tmp)

            @pl.loop(0, rows_per_tile // 16)
            def _(i):
                sl = pl.ds(i * 16, 16)
                accum[sl] = accum[sl] + tmp[sl]

        pltpu.sync_copy(accum,
                        out_hbm.at[pl.ds(c * n_pad + s * rows_per_tile,
                                         rows_per_tile)])

    return deg_kernel


def _sc_agg(n_pad, nch, d, k0=None, k1=None):
    if k0 is None:
        k0 = k1 = nch
    kmax = max(k0, k1)
    rows_per_tile = n_pad // NS
    mesh = plsc.VectorSubcoreMesh(core_axis_name="c", subcore_axis_name="s")

    @functools.partial(
        pl.kernel,
        out_type=jax.ShapeDtypeStruct((NC * n_pad, d), jnp.float32),
        mesh=mesh,
        scratch_types=[
            pltpu.VMEM((2, CH), jnp.int32),
            pltpu.VMEM((kmax, CH), jnp.int32),
            pltpu.VMEM((2, CH, d), jnp.float32),
            pltpu.VMEM_SHARED((n_pad, d), jnp.float32),
            pltpu.SemaphoreType.DMA((2,)),
        ],
    )
    def agg_kernel(hs_hbm, src_hbm, dst_hbm, zeros_hbm, out_hbm,
                   idx_s, idx_d, rows_v, acc, sems):
        c = lax.axis_index("c")
        s = lax.axis_index("s")
        wid = s * NC + c
        pltpu.sync_copy(zeros_hbm.at[pl.ds(s * rows_per_tile, rows_per_tile)],
                        acc.at[pl.ds(s * rows_per_tile, rows_per_tile)])
        pltpu.sync_copy(dst_hbm.at[wid], idx_d)
        plsc.subcore_barrier()

        pltpu.sync_copy(src_hbm.at[wid, 0], idx_s.at[0])
        pltpu.async_copy(hs_hbm.at[idx_s.at[0]], rows_v.at[0], sems.at[0])
        pltpu.sync_copy(src_hbm.at[wid, 1], idx_s.at[1])

        @pl.loop(0, nch - 1)
        def _(j):
            slot = lax.rem(j, 2)
            nslot = lax.rem(j + 1, 2)
            pltpu.make_async_copy(hs_hbm.at[idx_s.at[slot]], rows_v.at[slot],
                                  sems.at[slot]).wait()
            pltpu.async_copy(hs_hbm.at[idx_s.at[nslot]], rows_v.at[nslot],
                             sems.at[nslot])
            pltpu.sync_copy(rows_v.at[slot], acc.at[idx_d.at[j]], add=True)
            pltpu.sync_copy(src_hbm.at[wid, j + 2], idx_s.at[slot])

        last = lax.rem(nch - 1, 2)
        pltpu.make_async_copy(hs_hbm.at[idx_s.at[last]], rows_v.at[last],
                              sems.at[last]).wait()
        pltpu.sync_copy(rows_v.at[last], acc.at[idx_d.at[nch - 1]], add=True)

        plsc.subcore_barrier()
        pltpu.sync_copy(acc.at[pl.ds(s * rows_per_tile, rows_per_tile)],
                        out_hbm.at[pl.ds(c * n_pad + s * rows_per_tile,
                                         rows_per_tile)])

    return agg_kernel


def _tc_matmul(n_pad, f, k, bm):
    def body(x_ref, w_ref, b_ref, out_ref):
        out_ref[...] = jnp.dot(x_ref[...], w_ref[...],
                               preferred_element_type=jnp.float32) + b_ref[...]

    grid = (n_pad // bm,)
    return pl.pallas_call(
        body,
        grid=grid,
        in_specs=[
            pl.BlockSpec((bm, f), lambda i: (i, 0)),
            pl.BlockSpec((f, k), lambda i: (0, 0)),
            pl.BlockSpec((1, k), lambda i: (0, 0)),
        ],
        out_specs=pl.BlockSpec((bm, k), lambda i: (i, 0)),
        out_shape=jax.ShapeDtypeStruct((n_pad, k), jnp.float32),
    )


def _tc_scale(n_pad, h, bm):
    def body(h1_ref, p0_ref, p1_ref, hs_ref, dis_ref):
        deg = p0_ref[:, 0:1] + p1_ref[:, 0:1] + 1.0
        dis = lax.rsqrt(jnp.maximum(deg, 1.0))
        hs_ref[...] = h1_ref[...] * dis
        dis_ref[...] = jnp.broadcast_to(dis, (bm, 8))

    grid = (n_pad // bm,)
    return pl.pallas_call(
        body,
        grid=grid,
        in_specs=[
            pl.BlockSpec((bm, h), lambda i: (i, 0)),
            pl.BlockSpec((bm, 8), lambda i: (i, 0)),
            pl.BlockSpec((bm, 8), lambda i: (i, 0)),
        ],
        out_specs=[
            pl.BlockSpec((bm, h), lambda i: (i, 0)),
            pl.BlockSpec((bm, 8), lambda i: (i, 0)),
        ],
        out_shape=[
            jax.ShapeDtypeStruct((n_pad, h), jnp.float32),
            jax.ShapeDtypeStruct((n_pad, 8), jnp.float32),
        ],
    )


def _tc_layer2(n_pad, h, bm):
    def body(hs_ref, p0_ref, p1_ref, dis_ref, b1_ref, w_ref, out_ref):
        dis = dis_ref[:, 0:1]
        agg = (p0_ref[...] + p1_ref[...] + hs_ref[...]) * dis + b1_ref[...]
        z1 = jnp.maximum(agg, 0.0)
        out_ref[...] = jnp.dot(z1, w_ref[...],
                               preferred_element_type=jnp.float32) * dis

    grid = (n_pad // bm,)
    return pl.pallas_call(
        body,
        grid=grid,
        in_specs=[
            pl.BlockSpec((bm, h), lambda i: (i, 0)),
            pl.BlockSpec((bm, h), lambda i: (i, 0)),
            pl.BlockSpec((bm, h), lambda i: (i, 0)),
            pl.BlockSpec((bm, 8), lambda i: (i, 0)),
            pl.BlockSpec((1, h), lambda i: (0, 0)),
            pl.BlockSpec((h, h), lambda i: (0, 0)),
        ],
        out_specs=pl.BlockSpec((bm, h), lambda i: (i, 0)),
        out_shape=jax.ShapeDtypeStruct((n_pad, h), jnp.float32),
    )


def _tc_gru_head(n_pad, h, bm):
    def body(hs_ref, p0_ref, p1_ref, dis_ref, bvec_ref, h0_ref,
             wih_ref, gh_ref, bih_ref, whd_ref, bhd_ref,
             hn_ref, th_ref):
        dis = dis_ref[:, 0:1]
        z = (p0_ref[...] + p1_ref[...] + hs_ref[...]) * dis + bvec_ref[...]
        h0 = h0_ref[...]
        gi = jnp.dot(z, wih_ref[...],
                     preferred_element_type=jnp.float32) + bih_ref[...]
        gh = gh_ref[...]
        r = jax.nn.sigmoid(gi[:, :h] + gh[:, :h])
        zz = jax.nn.sigmoid(gi[:, h:2 * h] + gh[:, h:2 * h])
        nn_ = jnp.tanh(gi[:, 2 * h:] + r * gh[:, 2 * h:])
        h_next = (1.0 - zz) * nn_ + zz * h0
        hn_ref[...] = h_next
        th_ref[...] = jnp.dot(h_next, whd_ref[...],
                              preferred_element_type=jnp.float32) + bhd_ref[...]

    grid = (n_pad // bm,)
    return pl.pallas_call(
        body,
        grid=grid,
        in_specs=[
            pl.BlockSpec((bm, h), lambda i: (i, 0)),
            pl.BlockSpec((bm, h), lambda i: (i, 0)),
            pl.BlockSpec((bm, h), lambda i: (i, 0)),
            pl.BlockSpec((bm, 8), lambda i: (i, 0)),
            pl.BlockSpec((1, h), lambda i: (0, 0)),
            pl.BlockSpec((bm, h), lambda i: (i, 0)),
            pl.BlockSpec((h, 3 * h), lambda i: (0, 0)),
            pl.BlockSpec((bm, 3 * h), lambda i: (i, 0)),
            pl.BlockSpec((1, 3 * h), lambda i: (0, 0)),
            pl.BlockSpec((h, 8), lambda i: (0, 0)),
            pl.BlockSpec((1, 8), lambda i: (0, 0)),
        ],
        out_specs=[
            pl.BlockSpec((bm, h), lambda i: (i, 0)),
            pl.BlockSpec((bm, 8), lambda i: (i, 0)),
        ],
        out_shape=[
            jax.ShapeDtypeStruct((n_pad, h), jnp.float32),
            jax.ShapeDtypeStruct((n_pad, 8), jnp.float32),
        ],
    )


@jax.jit
def _run(g, x, t_over_t, h_prev, W1, b1, W2, b2, Wt, bt,
         W_ih, W_hh, b_ih, b_hh, W_head, b_head):
    n, f = x.shape
    h = W1.shape[1]
    e = g.shape[1]

    n_pad = ((n + 1 + 1023) // 1024) * 1024
    ce = ((e + NW - 1) // NW + CH - 1) // CH * CH
    nch = ce // CH
    e_pad = ce * NW

    src = g[0].astype(jnp.int32)
    dst = g[1].astype(jnp.int32)
    fill = jnp.full((e_pad - e,), n, dtype=jnp.int32)
    src_p = jnp.concatenate([src, fill])
    dst_p = jnp.concatenate([dst, fill])
    dst_even = dst_p.reshape(NW, ce)
    k0 = k1 = nch

    src_r = src_p.reshape(NW, nch, CH)
    src_r = jnp.concatenate(
        [src_r, jnp.full((NW, 1, CH), n, dtype=jnp.int32)], axis=1)
    dst_r = dst_p.reshape(NW, nch, CH)

    x_pad = jnp.zeros((n_pad, f), x.dtype).at[:n].set(x)
    h0_pad = jnp.zeros((n_pad, h), x.dtype).at[:n].set(h_prev[0])
    zeros_big = jnp.zeros((n_pad, h), jnp.float32)

    deg_parts = _sc_deg(n_pad, ce)(dst_even)
    dp0 = jnp.broadcast_to(deg_parts[:n_pad, None], (n_pad, 8))
    dp1 = jnp.broadcast_to(deg_parts[n_pad:, None], (n_pad, 8))

    zero_h = jnp.zeros((1, h), jnp.float32)
    h1 = _tc_matmul(n_pad, f, h, 1024)(x_pad, W1, zero_h)
    hs1, dis = _tc_scale(n_pad, h, 1024)(h1, dp0, dp1)

    agg = _sc_agg(n_pad, nch, h, k0, k1)
    parts1 = agg(hs1, src_r, dst_r, zeros_big)
    hs2 = _tc_layer2(n_pad, h, 1024)(
        hs1, parts1[:n_pad], parts1[n_pad:], dis, b1.reshape(1, h), W2)

    parts2 = agg(hs2, src_r, dst_r, zeros_big)

    gh = _tc_matmul(n_pad, h, 3 * h, 1024)(h0_pad, W_hh.T,
                                           b_hh.reshape(1, 3 * h))

    bvec = (b2 + bt + t_over_t[0] * Wt[:, 0]).reshape(1, h)
    whd = jnp.zeros((h, 8), jnp.float32).at[:, :3].set(W_head.T)
    bhd = jnp.zeros((1, 8), jnp.float32).at[0, :3].set(b_head)
    h_next_pad, theta_pad = _tc_gru_head(n_pad, h, 1024)(
        hs2, parts2[:n_pad], parts2[n_pad:], dis, bvec, h0_pad,
        W_ih.T, gh, b_ih.reshape(1, 3 * h),
        whd, bhd)

    return theta_pad[:n, :3], h_next_pad[:n]


def kernel(g, x, t_over_t, h_prev, W1, b1, W2, b2, Wt, bt,
           W_ih, W_hh, b_ih, b_hh, W_head, b_head):
    return _run(g, x, t_over_t, h_prev, W1, b1, W2, b2, Wt, bt,
                W_ih, W_hh, b_ih, b_hh, W_head, b_head)
